# baseline (speedup 1.0000x reference)
import functools

import jax
import jax.numpy as jnp
from jax.experimental import pallas as pl
from jax.experimental.pallas import tpu as pltpu

CH = 512


def _body(idx_ref, table_ref, code_ref, out_ref, carry_ref):
    k = pl.program_id(1)
    nk = pl.num_programs(1) - 1
    cur = code_ref[0]
    lang_row = table_ref[idx_ref[0]]
    first = jnp.where(k == 0, lang_row, carry_ref[0])[None]
    out_ref[0] = jnp.concatenate([first, cur[: CH - 1]], axis=0)
    carry_ref[0] = cur[CH - 1]


def kernel(code_embeddings, language_embed, lang_idx):
    B, S, D = code_embeddings.shape
    nk = S // CH
    assert nk * CH == S

    grid_spec = pltpu.PrefetchScalarGridSpec(
        num_scalar_prefetch=1,
        grid=(B, nk + 1),
        in_specs=[
            pl.BlockSpec((language_embed.shape[0], D), lambda b, k, idx: (0, 0)),
            pl.BlockSpec((1, CH, D), lambda b, k, idx: (b, jnp.minimum(k, nk - 1), 0)),
        ],
        out_specs=pl.BlockSpec((1, CH, D), lambda b, k, idx: (b, k, 0)),
        scratch_shapes=[pltpu.VMEM((1, D), code_embeddings.dtype)],
    )
    return pl.pallas_call(
        _body,
        grid_spec=grid_spec,
        out_shape=jax.ShapeDtypeStruct((B, S + 1, D), code_embeddings.dtype),
        compiler_params=pltpu.CompilerParams(
            dimension_semantics=("arbitrary", "arbitrary"),
        ),
    )(lang_idx, language_embed, code_embeddings)


# speedup vs baseline: 1.0479x; 1.0479x over previous
import functools

import jax
import jax.numpy as jnp
from jax.experimental import pallas as pl
from jax.experimental.pallas import tpu as pltpu

CH = 1024


def _body(idx_ref, table_ref, code_ref, out_ref, carry_ref):
    k = pl.program_id(1)
    nk = pl.num_programs(1) - 1
    cur = code_ref[0]
    lang_row = table_ref[idx_ref[0]]
    first = jnp.where(k == 0, lang_row, carry_ref[0])[None]
    out_ref[0] = jnp.concatenate([first, cur[: CH - 1]], axis=0)
    carry_ref[0] = cur[CH - 1]


def kernel(code_embeddings, language_embed, lang_idx):
    B, S, D = code_embeddings.shape
    nk = S // CH
    assert nk * CH == S

    grid_spec = pltpu.PrefetchScalarGridSpec(
        num_scalar_prefetch=1,
        grid=(B, nk + 1),
        in_specs=[
            pl.BlockSpec((language_embed.shape[0], D), lambda b, k, idx: (0, 0)),
            pl.BlockSpec((1, CH, D), lambda b, k, idx: (b, jnp.minimum(k, nk - 1), 0)),
        ],
        out_specs=pl.BlockSpec((1, CH, D), lambda b, k, idx: (b, k, 0)),
        scratch_shapes=[pltpu.VMEM((1, D), code_embeddings.dtype)],
    )
    return pl.pallas_call(
        _body,
        grid_spec=grid_spec,
        out_shape=jax.ShapeDtypeStruct((B, S + 1, D), code_embeddings.dtype),
        compiler_params=pltpu.CompilerParams(
            dimension_semantics=("arbitrary", "arbitrary"),
        ),
    )(lang_idx, language_embed, code_embeddings)


# speedup vs baseline: 3.5706x; 3.4075x over previous
import jax
import jax.numpy as jnp
from jax.experimental import pallas as pl
from jax.experimental.pallas import tpu as pltpu

CH = 512
NBUF = 2


def kernel(code_embeddings, language_embed, lang_idx):
    B, S, D = code_embeddings.shape
    NK = S // CH
    assert NK * CH == S

    def body(idx_ref, table_ref, code_hbm, out_hbm,
             in_bufs, out_bufs, slab0, in_sems, out_sems, slab0_sem):
        lang = table_ref[idx_ref[0]]
        slab0[...] = jnp.broadcast_to(lang[None, None], (1, B, D))
        slab0_cp = pltpu.make_async_copy(slab0, out_hbm.at[pl.ds(0, 1)], slab0_sem)
        slab0_cp.start()

        def in_copy(k, slot, b):
            return pltpu.make_async_copy(
                code_hbm.at[b, pl.ds(k * CH, CH)], in_bufs.at[slot, b], in_sems.at[slot])

        def out_copy(k, slot):
            return pltpu.make_async_copy(
                out_bufs.at[slot], out_hbm.at[pl.ds(1 + k * CH, CH)], out_sems.at[slot])

        for k in range(min(NBUF, NK)):
            for b in range(B):
                in_copy(k, k % NBUF, b).start()

        for k in range(NK):
            slot = k % NBUF
            for b in range(B):
                in_copy(k, slot, b).wait()
            if k >= NBUF:
                out_copy(k - NBUF, slot).wait()
            x = in_bufs[slot]
            out_bufs[slot] = x.transpose(1, 0, 2)
            out_copy(k, slot).start()
            if k + NBUF < NK:
                for b in range(B):
                    in_copy(k + NBUF, slot, b).start()

        for k in range(max(NK - NBUF, 0), NK):
            out_copy(k, k % NBUF).wait()
        slab0_cp.wait()

    out_sm = pl.pallas_call(
        body,
        in_specs=[
            pl.BlockSpec(memory_space=pltpu.SMEM),
            pl.BlockSpec(memory_space=pltpu.VMEM),
            pl.BlockSpec(memory_space=pl.ANY),
        ],
        out_specs=pl.BlockSpec(memory_space=pl.ANY),
        out_shape=jax.ShapeDtypeStruct((S + 1, B, D), code_embeddings.dtype),
        compiler_params=pltpu.CompilerParams(vmem_limit_bytes=60 * 1024 * 1024),
        scratch_shapes=[
            pltpu.VMEM((NBUF, B, CH, D), code_embeddings.dtype),
            pltpu.VMEM((NBUF, CH, B, D), code_embeddings.dtype),
            pltpu.VMEM((1, B, D), code_embeddings.dtype),
            pltpu.SemaphoreType.DMA((NBUF,)),
            pltpu.SemaphoreType.DMA((NBUF,)),
            pltpu.SemaphoreType.DMA,
        ],
    )(lang_idx, language_embed, code_embeddings)

    return jnp.swapaxes(out_sm, 0, 1)


# speedup vs baseline: 3.6551x; 1.0237x over previous
import jax
import jax.numpy as jnp
from jax.experimental import pallas as pl
from jax.experimental.pallas import tpu as pltpu

CH = 1024
NBUF = 2


def kernel(code_embeddings, language_embed, lang_idx):
    B, S, D = code_embeddings.shape
    NK = S // CH
    assert NK * CH == S

    def body(idx_ref, table_ref, code_hbm, out_hbm,
             in_bufs, out_bufs, slab0, in_sems, out_sems, slab0_sem):
        lang = table_ref[idx_ref[0]]
        slab0[...] = jnp.broadcast_to(lang[None, None], (1, B, D))
        slab0_cp = pltpu.make_async_copy(slab0, out_hbm.at[pl.ds(0, 1)], slab0_sem)
        slab0_cp.start()

        def in_copy(k, slot, b):
            return pltpu.make_async_copy(
                code_hbm.at[b, pl.ds(k * CH, CH)], in_bufs.at[slot, b], in_sems.at[slot])

        def out_copy(k, slot):
            return pltpu.make_async_copy(
                out_bufs.at[slot], out_hbm.at[pl.ds(1 + k * CH, CH)], out_sems.at[slot])

        for k in range(min(NBUF, NK)):
            for b in range(B):
                in_copy(k, k % NBUF, b).start()

        for k in range(NK):
            slot = k % NBUF
            for b in range(B):
                in_copy(k, slot, b).wait()
            if k >= NBUF:
                out_copy(k - NBUF, slot).wait()
            x = in_bufs[slot]
            out_bufs[slot] = x.transpose(1, 0, 2)
            out_copy(k, slot).start()
            if k + NBUF < NK:
                for b in range(B):
                    in_copy(k + NBUF, slot, b).start()

        for k in range(max(NK - NBUF, 0), NK):
            out_copy(k, k % NBUF).wait()
        slab0_cp.wait()

    out_sm = pl.pallas_call(
        body,
        in_specs=[
            pl.BlockSpec(memory_space=pltpu.SMEM),
            pl.BlockSpec(memory_space=pltpu.VMEM),
            pl.BlockSpec(memory_space=pl.ANY),
        ],
        out_specs=pl.BlockSpec(memory_space=pl.ANY),
        out_shape=jax.ShapeDtypeStruct((S + 1, B, D), code_embeddings.dtype),
        compiler_params=pltpu.CompilerParams(vmem_limit_bytes=60 * 1024 * 1024),
        scratch_shapes=[
            pltpu.VMEM((NBUF, B, CH, D), code_embeddings.dtype),
            pltpu.VMEM((NBUF, CH, B, D), code_embeddings.dtype),
            pltpu.VMEM((1, B, D), code_embeddings.dtype),
            pltpu.SemaphoreType.DMA((NBUF,)),
            pltpu.SemaphoreType.DMA((NBUF,)),
            pltpu.SemaphoreType.DMA,
        ],
    )(lang_idx, language_embed, code_embeddings)

    return jnp.swapaxes(out_sm, 0, 1)
